# baseline (speedup 1.0000x reference)
import jax
import jax.numpy as jnp
from jax import lax
from jax.experimental import pallas as pl
from jax.experimental.pallas import tpu as pltpu
from jax.experimental.pallas import tpu_sc as plsc

B = 4096
D = 64
K = 100000
H = 2 * D
G = 3 * H
EMA_M = 0.5

NC = 2
NS = 16
NW = NC * NS
BPW = B // NW

TILE = 512


def _sc_mesh():
    return plsc.VectorSubcoreMesh(
        core_axis_name="c", subcore_axis_name="s",
        num_cores=NC, num_subcores=NS)


def _gather_body(a_hbm, idx_hbm, old_hbm, idx_v, rows_v, sem):
    wid = lax.axis_index("s") * NC + lax.axis_index("c")
    base = wid * BPW
    pltpu.sync_copy(idx_hbm.at[pl.ds(base, BPW)], idx_v)
    pltpu.async_copy(a_hbm.at[idx_v], rows_v, sem).wait()
    pltpu.sync_copy(rows_v, old_hbm.at[pl.ds(base, BPW)])


def _sc_gather(a, flat_idx):
    return pl.kernel(
        _gather_body,
        out_type=jax.ShapeDtypeStruct((B, H), jnp.float32),
        mesh=_sc_mesh(),
        scratch_types=[
            pltpu.VMEM((BPW,), jnp.int32),
            pltpu.VMEM((BPW, H), jnp.float32),
            pltpu.SemaphoreType.DMA,
        ],
    )(a, flat_idx)


def _scatter_body(out_hbm, new_hbm, idx_hbm, idx_v, rows_v, sem):
    wid = lax.axis_index("s") * NC + lax.axis_index("c")
    base = wid * BPW
    pltpu.sync_copy(idx_hbm.at[pl.ds(base, BPW)], idx_v)
    pltpu.sync_copy(new_hbm.at[pl.ds(base, BPW)], rows_v)
    pltpu.async_copy(rows_v, out_hbm.at[idx_v], sem).wait()


def _sc_scatter(out_ref, new, flat_idx):
    return pl.kernel(
        _scatter_body,
        out_type=(),
        mesh=_sc_mesh(),
        scratch_types=[
            pltpu.VMEM((BPW,), jnp.int32),
            pltpu.VMEM((BPW, H), jnp.float32),
            pltpu.SemaphoreType.DMA,
        ],
    )(out_ref, new, flat_idx)


CPW = 3128
CPW_LAST = K - (NW - 1) * CPW


def _copy_sc_body(a_hbm, out_hbm, sem):
    wid = lax.axis_index("s") * NC + lax.axis_index("c")
    base = wid * CPW

    @pl.when(wid < NW - 1)
    def _():
        pltpu.async_copy(a_hbm.at[pl.ds(base, CPW)],
                         out_hbm.at[pl.ds(base, CPW)], sem).wait()

    @pl.when(wid == NW - 1)
    def _():
        lo = (NW - 1) * CPW
        pltpu.async_copy(a_hbm.at[pl.ds(lo, CPW_LAST)],
                         out_hbm.at[pl.ds(lo, CPW_LAST)], sem).wait()


def _sc_copy(a):
    return pl.kernel(
        _copy_sc_body,
        out_type=jax.ShapeDtypeStruct((K, H), jnp.float32),
        mesh=_sc_mesh(),
        scratch_types=[pltpu.SemaphoreType.DMA],
    )(a)


def _compute_body(idxc_ref, idxr_ref, hf_ref, hs_ref, old_ref,
                  wih_ref, whh_ref, bih_ref, bhh_ref, out_ref):
    eq = idxc_ref[...] == idxr_ref[...]
    e = eq.astype(jnp.bfloat16)
    cnt = jnp.sum(eq.astype(jnp.float32), axis=1, keepdims=True)
    dn = (((1,), (0,)), ((), ()))
    msf = lax.dot_general(e, hf_ref[...], dn,
                          preferred_element_type=jnp.float32)
    mss = lax.dot_general(e, hs_ref[...], dn,
                          preferred_element_type=jnp.float32)
    scale = 1.0 / (cnt + 1e-9)
    m = jnp.concatenate([msf * scale, mss * scale], axis=1)

    old = old_ref[...]
    dnt = (((1,), (1,)), ((), ()))
    gi = lax.dot_general(m, wih_ref[...], dnt) + bih_ref[...]
    gh = lax.dot_general(old, whh_ref[...], dnt) + bhh_ref[...]
    r = jax.nn.sigmoid(gi[:, :H] + gh[:, :H])
    z = jax.nn.sigmoid(gi[:, H:2 * H] + gh[:, H:2 * H])
    n = jnp.tanh(gi[:, 2 * H:] + r * gh[:, 2 * H:])
    hn = (1.0 - z) * n + z * old
    out_ref[...] = old * (1.0 - EMA_M) + hn * EMA_M


def _tc_compute(idx_col, idx_row, h_fast, h_slow, old, W_ih, W_hh, b_ih, b_hh):
    grid = (B // TILE,)
    return pl.pallas_call(
        _compute_body,
        grid=grid,
        in_specs=[
            pl.BlockSpec((TILE, 1), lambda i: (i, 0)),
            pl.BlockSpec((1, B), lambda i: (0, 0)),
            pl.BlockSpec((B, D), lambda i: (0, 0)),
            pl.BlockSpec((B, D), lambda i: (0, 0)),
            pl.BlockSpec((TILE, H), lambda i: (i, 0)),
            pl.BlockSpec((G, H), lambda i: (0, 0)),
            pl.BlockSpec((G, H), lambda i: (0, 0)),
            pl.BlockSpec((1, G), lambda i: (0, 0)),
            pl.BlockSpec((1, G), lambda i: (0, 0)),
        ],
        out_specs=pl.BlockSpec((TILE, H), lambda i: (i, 0)),
        out_shape=jax.ShapeDtypeStruct((B, H), jnp.float32),
        compiler_params=pltpu.CompilerParams(
            dimension_semantics=("parallel",),
        ),
    )(idx_col, idx_row, h_fast, h_slow, old, W_ih, W_hh, b_ih, b_hh)


def kernel(h_fast, h_slow, idx, A_states, W_ih, W_hh, b_ih, b_hh):
    flat_idx = idx.reshape(-1)
    idx_col = idx.reshape(B, 1)
    idx_row = idx.reshape(1, B)
    old = _sc_gather(A_states, flat_idx)
    new = _tc_compute(idx_col, idx_row,
                      h_fast.astype(jnp.bfloat16), h_slow.astype(jnp.bfloat16),
                      old, W_ih, W_hh, b_ih.reshape(1, G), b_hh.reshape(1, G))
    out_ref = jax.new_ref(_sc_copy(A_states))
    _sc_scatter(out_ref, new, flat_idx)
    return out_ref[...]


# speedup vs baseline: 19.5157x; 19.5157x over previous
import jax
import jax.numpy as jnp
from jax import lax
from jax.experimental import pallas as pl
from jax.experimental.pallas import tpu as pltpu
from jax.experimental.pallas import tpu_sc as plsc

B = 4096
D = 64
K = 100000
H = 2 * D
G = 3 * H
EMA_M = 0.5

NC = 2
NS = 16
NW = NC * NS
BPW = B // NW

TILE = 512


def _sc_mesh():
    return plsc.VectorSubcoreMesh(
        core_axis_name="c", subcore_axis_name="s",
        num_cores=NC, num_subcores=NS)


def _gather_body(a_hbm, idx_hbm, old_hbm, idx_v, rows_v, sem):
    wid = lax.axis_index("s") * NC + lax.axis_index("c")
    base = wid * BPW
    pltpu.sync_copy(idx_hbm.at[pl.ds(base, BPW)], idx_v)
    pltpu.async_copy(a_hbm.at[idx_v], rows_v, sem).wait()
    pltpu.sync_copy(rows_v, old_hbm.at[pl.ds(base, BPW)])


def _sc_gather(a, flat_idx):
    return pl.kernel(
        _gather_body,
        out_type=jax.ShapeDtypeStruct((B, H), jnp.float32),
        mesh=_sc_mesh(),
        scratch_types=[
            pltpu.VMEM((BPW,), jnp.int32),
            pltpu.VMEM((BPW, H), jnp.float32),
            pltpu.SemaphoreType.DMA,
        ],
    )(a, flat_idx)


def _scatter_body(out_hbm, new_hbm, idx_hbm, idx_v, rows_v, sem):
    wid = lax.axis_index("s") * NC + lax.axis_index("c")
    base = wid * BPW
    pltpu.sync_copy(idx_hbm.at[pl.ds(base, BPW)], idx_v)
    pltpu.sync_copy(new_hbm.at[pl.ds(base, BPW)], rows_v)
    pltpu.async_copy(rows_v, out_hbm.at[idx_v], sem).wait()


def _sc_scatter(out_ref, new, flat_idx):
    return pl.kernel(
        _scatter_body,
        out_type=(),
        mesh=_sc_mesh(),
        scratch_types=[
            pltpu.VMEM((BPW,), jnp.int32),
            pltpu.VMEM((BPW, H), jnp.float32),
            pltpu.SemaphoreType.DMA,
        ],
    )(out_ref, new, flat_idx)


NSTEP = 20
CROWS = K // NSTEP
CTILE = 208
BP = NSTEP * CTILE


def _fused_body(a_ref, idxc_ref, idxr_ref, hf_ref, hs_ref, old_ref,
                wih_ref, whh_ref, bih_ref, bhh_ref, cp_ref, new_ref):
    cp_ref[...] = a_ref[...]

    eq = idxc_ref[...] == idxr_ref[...]
    e = eq.astype(jnp.bfloat16)
    cnt = jnp.sum(eq.astype(jnp.float32), axis=1, keepdims=True)
    dn = (((1,), (0,)), ((), ()))
    msf = lax.dot_general(e, hf_ref[...], dn,
                          preferred_element_type=jnp.float32)
    mss = lax.dot_general(e, hs_ref[...], dn,
                          preferred_element_type=jnp.float32)
    scale = 1.0 / (cnt + 1e-9)
    m = jnp.concatenate([msf * scale, mss * scale], axis=1)

    old = old_ref[...]
    dnt = (((1,), (1,)), ((), ()))
    gi = lax.dot_general(m, wih_ref[...], dnt) + bih_ref[...]
    gh = lax.dot_general(old, whh_ref[...], dnt) + bhh_ref[...]
    r = jax.nn.sigmoid(gi[:, :H] + gh[:, :H])
    z = jax.nn.sigmoid(gi[:, H:2 * H] + gh[:, H:2 * H])
    n = jnp.tanh(gi[:, 2 * H:] + r * gh[:, 2 * H:])
    hn = (1.0 - z) * n + z * old
    new_ref[...] = old * (1.0 - EMA_M) + hn * EMA_M


def _tc_fused(a, idx_col_pad, idx_row, h_fast, h_slow, old_pad,
              W_ih, W_hh, b_ih, b_hh):
    return pl.pallas_call(
        _fused_body,
        grid=(NSTEP,),
        in_specs=[
            pl.BlockSpec((CROWS, H), lambda i: (i, 0)),
            pl.BlockSpec((CTILE, 1), lambda i: (i, 0)),
            pl.BlockSpec((1, B), lambda i: (0, 0)),
            pl.BlockSpec((B, D), lambda i: (0, 0)),
            pl.BlockSpec((B, D), lambda i: (0, 0)),
            pl.BlockSpec((CTILE, H), lambda i: (i, 0)),
            pl.BlockSpec((G, H), lambda i: (0, 0)),
            pl.BlockSpec((G, H), lambda i: (0, 0)),
            pl.BlockSpec((1, G), lambda i: (0, 0)),
            pl.BlockSpec((1, G), lambda i: (0, 0)),
        ],
        out_specs=[
            pl.BlockSpec((CROWS, H), lambda i: (i, 0)),
            pl.BlockSpec((CTILE, H), lambda i: (i, 0)),
        ],
        out_shape=[
            jax.ShapeDtypeStruct((K, H), jnp.float32),
            jax.ShapeDtypeStruct((BP, H), jnp.float32),
        ],
        compiler_params=pltpu.CompilerParams(
            dimension_semantics=("parallel",),
        ),
    )(a, idx_col_pad, idx_row, h_fast, h_slow, old_pad,
      W_ih, W_hh, b_ih, b_hh)


def _compute_body(idxc_ref, idxr_ref, hf_ref, hs_ref, old_ref,
                  wih_ref, whh_ref, bih_ref, bhh_ref, out_ref):
    eq = idxc_ref[...] == idxr_ref[...]
    e = eq.astype(jnp.bfloat16)
    cnt = jnp.sum(eq.astype(jnp.float32), axis=1, keepdims=True)
    dn = (((1,), (0,)), ((), ()))
    msf = lax.dot_general(e, hf_ref[...], dn,
                          preferred_element_type=jnp.float32)
    mss = lax.dot_general(e, hs_ref[...], dn,
                          preferred_element_type=jnp.float32)
    scale = 1.0 / (cnt + 1e-9)
    m = jnp.concatenate([msf * scale, mss * scale], axis=1)

    old = old_ref[...]
    dnt = (((1,), (1,)), ((), ()))
    gi = lax.dot_general(m, wih_ref[...], dnt) + bih_ref[...]
    gh = lax.dot_general(old, whh_ref[...], dnt) + bhh_ref[...]
    r = jax.nn.sigmoid(gi[:, :H] + gh[:, :H])
    z = jax.nn.sigmoid(gi[:, H:2 * H] + gh[:, H:2 * H])
    n = jnp.tanh(gi[:, 2 * H:] + r * gh[:, 2 * H:])
    hn = (1.0 - z) * n + z * old
    out_ref[...] = old * (1.0 - EMA_M) + hn * EMA_M


def _tc_compute(idx_col, idx_row, h_fast, h_slow, old, W_ih, W_hh, b_ih, b_hh):
    grid = (B // TILE,)
    return pl.pallas_call(
        _compute_body,
        grid=grid,
        in_specs=[
            pl.BlockSpec((TILE, 1), lambda i: (i, 0)),
            pl.BlockSpec((1, B), lambda i: (0, 0)),
            pl.BlockSpec((B, D), lambda i: (0, 0)),
            pl.BlockSpec((B, D), lambda i: (0, 0)),
            pl.BlockSpec((TILE, H), lambda i: (i, 0)),
            pl.BlockSpec((G, H), lambda i: (0, 0)),
            pl.BlockSpec((G, H), lambda i: (0, 0)),
            pl.BlockSpec((1, G), lambda i: (0, 0)),
            pl.BlockSpec((1, G), lambda i: (0, 0)),
        ],
        out_specs=pl.BlockSpec((TILE, H), lambda i: (i, 0)),
        out_shape=jax.ShapeDtypeStruct((B, H), jnp.float32),
        compiler_params=pltpu.CompilerParams(
            dimension_semantics=("parallel",),
        ),
    )(idx_col, idx_row, h_fast, h_slow, old, W_ih, W_hh, b_ih, b_hh)


def kernel(h_fast, h_slow, idx, A_states, W_ih, W_hh, b_ih, b_hh):
    flat_idx = idx.reshape(-1)
    idx_col_pad = jnp.concatenate(
        [idx.reshape(B, 1), jnp.zeros((BP - B, 1), jnp.int32)])
    idx_row = idx.reshape(1, B)
    old = _sc_gather(A_states, flat_idx)
    old_pad = jnp.concatenate([old, jnp.zeros((BP - B, H), jnp.float32)])
    cp, new_pad = _tc_fused(
        A_states, idx_col_pad, idx_row,
        h_fast.astype(jnp.bfloat16), h_slow.astype(jnp.bfloat16),
        old_pad, W_ih, W_hh, b_ih.reshape(1, G), b_hh.reshape(1, G))
    out_ref = jax.new_ref(cp)
    _sc_scatter(out_ref, new_pad, flat_idx)
    return out_ref[...]


# speedup vs baseline: 20.3790x; 1.0442x over previous
import jax
import jax.numpy as jnp
from jax import lax
from jax.experimental import pallas as pl
from jax.experimental.pallas import tpu as pltpu
from jax.experimental.pallas import tpu_sc as plsc

B = 4096
D = 64
K = 100000
H = 2 * D
G = 3 * H
EMA_M = 0.5

NC = 2
NS = 16
NW = NC * NS
BPW = B // NW

TILE = 512


def _sc_mesh():
    return plsc.VectorSubcoreMesh(
        core_axis_name="c", subcore_axis_name="s",
        num_cores=NC, num_subcores=NS)


def _gather_body(a_hbm, idx_hbm, old_hbm, idx_v, rows_v, sem):
    wid = lax.axis_index("s") * NC + lax.axis_index("c")
    base = wid * BPW
    pltpu.sync_copy(idx_hbm.at[pl.ds(base, BPW)], idx_v)
    pltpu.async_copy(a_hbm.at[idx_v], rows_v, sem).wait()
    pltpu.sync_copy(rows_v, old_hbm.at[pl.ds(base, BPW)])


def _sc_gather(a, flat_idx):
    return pl.kernel(
        _gather_body,
        out_type=jax.ShapeDtypeStruct((B, H), jnp.float32),
        mesh=_sc_mesh(),
        scratch_types=[
            pltpu.VMEM((BPW,), jnp.int32),
            pltpu.VMEM((BPW, H), jnp.float32),
            pltpu.SemaphoreType.DMA,
        ],
    )(a, flat_idx)


def _scatter_body(out_hbm, new_hbm, idx_hbm, idx_v, rows_v, sem):
    wid = lax.axis_index("s") * NC + lax.axis_index("c")
    base = wid * BPW
    pltpu.sync_copy(idx_hbm.at[pl.ds(base, BPW)], idx_v)
    pltpu.sync_copy(new_hbm.at[pl.ds(base, BPW)], rows_v)
    pltpu.async_copy(rows_v, out_hbm.at[idx_v], sem).wait()


def _sc_scatter(out_ref, new, flat_idx):
    return pl.kernel(
        _scatter_body,
        out_type=(),
        mesh=_sc_mesh(),
        scratch_types=[
            pltpu.VMEM((BPW,), jnp.int32),
            pltpu.VMEM((BPW, H), jnp.float32),
            pltpu.SemaphoreType.DMA,
        ],
    )(out_ref, new, flat_idx)


NSTEP = 20
CROWS = K // NSTEP
CTILE = 208
BP = NSTEP * CTILE


def _fused_body(a_ref, idxc_ref, idxr_ref, hf_ref, hs_ref, old_ref,
                wih_ref, whh_ref, bih_ref, bhh_ref, cp_ref, new_ref):
    cp_ref[...] = a_ref[...]

    eq = idxc_ref[...] == idxr_ref[...]
    e = eq.astype(jnp.bfloat16)
    cnt = jnp.sum(eq.astype(jnp.float32), axis=1, keepdims=True)
    dn = (((1,), (0,)), ((), ()))
    msf = lax.dot_general(e, hf_ref[...], dn,
                          preferred_element_type=jnp.float32)
    mss = lax.dot_general(e, hs_ref[...], dn,
                          preferred_element_type=jnp.float32)
    scale = 1.0 / (cnt + 1e-9)
    m = jnp.concatenate([msf * scale, mss * scale], axis=1)

    old = old_ref[...]
    dnt = (((1,), (1,)), ((), ()))
    gi = lax.dot_general(m, wih_ref[...], dnt) + bih_ref[...]
    gh = lax.dot_general(old, whh_ref[...], dnt) + bhh_ref[...]
    r = jax.nn.sigmoid(gi[:, :H] + gh[:, :H])
    z = jax.nn.sigmoid(gi[:, H:2 * H] + gh[:, H:2 * H])
    n = jnp.tanh(gi[:, 2 * H:] + r * gh[:, 2 * H:])
    hn = (1.0 - z) * n + z * old
    new_ref[...] = old * (1.0 - EMA_M) + hn * EMA_M


def _tc_fused(a, idx_col_pad, idx_row, h_fast, h_slow, old_pad,
              W_ih, W_hh, b_ih, b_hh):
    return pl.pallas_call(
        _fused_body,
        grid=(NSTEP,),
        in_specs=[
            pl.BlockSpec((CROWS, H), lambda i: (i, 0)),
            pl.BlockSpec((CTILE, 1), lambda i: (i, 0)),
            pl.BlockSpec((1, B), lambda i: (0, 0)),
            pl.BlockSpec((B, D), lambda i: (0, 0)),
            pl.BlockSpec((B, D), lambda i: (0, 0)),
            pl.BlockSpec((CTILE, H), lambda i: (i, 0)),
            pl.BlockSpec((G, H), lambda i: (0, 0)),
            pl.BlockSpec((G, H), lambda i: (0, 0)),
            pl.BlockSpec((1, G), lambda i: (0, 0)),
            pl.BlockSpec((1, G), lambda i: (0, 0)),
        ],
        out_specs=[
            pl.BlockSpec((CROWS, H), lambda i: (i, 0)),
            pl.BlockSpec((CTILE, H), lambda i: (i, 0)),
        ],
        out_shape=[
            jax.ShapeDtypeStruct((K, H), jnp.float32),
            jax.ShapeDtypeStruct((B, H), jnp.float32),
        ],
        compiler_params=pltpu.CompilerParams(
            dimension_semantics=("parallel",),
        ),
    )(a, idx_col_pad, idx_row, h_fast, h_slow, old_pad,
      W_ih, W_hh, b_ih, b_hh)


def _compute_body(idxc_ref, idxr_ref, hf_ref, hs_ref, old_ref,
                  wih_ref, whh_ref, bih_ref, bhh_ref, out_ref):
    eq = idxc_ref[...] == idxr_ref[...]
    e = eq.astype(jnp.bfloat16)
    cnt = jnp.sum(eq.astype(jnp.float32), axis=1, keepdims=True)
    dn = (((1,), (0,)), ((), ()))
    msf = lax.dot_general(e, hf_ref[...], dn,
                          preferred_element_type=jnp.float32)
    mss = lax.dot_general(e, hs_ref[...], dn,
                          preferred_element_type=jnp.float32)
    scale = 1.0 / (cnt + 1e-9)
    m = jnp.concatenate([msf * scale, mss * scale], axis=1)

    old = old_ref[...]
    dnt = (((1,), (1,)), ((), ()))
    gi = lax.dot_general(m, wih_ref[...], dnt) + bih_ref[...]
    gh = lax.dot_general(old, whh_ref[...], dnt) + bhh_ref[...]
    r = jax.nn.sigmoid(gi[:, :H] + gh[:, :H])
    z = jax.nn.sigmoid(gi[:, H:2 * H] + gh[:, H:2 * H])
    n = jnp.tanh(gi[:, 2 * H:] + r * gh[:, 2 * H:])
    hn = (1.0 - z) * n + z * old
    out_ref[...] = old * (1.0 - EMA_M) + hn * EMA_M


def _tc_compute(idx_col, idx_row, h_fast, h_slow, old, W_ih, W_hh, b_ih, b_hh):
    grid = (B // TILE,)
    return pl.pallas_call(
        _compute_body,
        grid=grid,
        in_specs=[
            pl.BlockSpec((TILE, 1), lambda i: (i, 0)),
            pl.BlockSpec((1, B), lambda i: (0, 0)),
            pl.BlockSpec((B, D), lambda i: (0, 0)),
            pl.BlockSpec((B, D), lambda i: (0, 0)),
            pl.BlockSpec((TILE, H), lambda i: (i, 0)),
            pl.BlockSpec((G, H), lambda i: (0, 0)),
            pl.BlockSpec((G, H), lambda i: (0, 0)),
            pl.BlockSpec((1, G), lambda i: (0, 0)),
            pl.BlockSpec((1, G), lambda i: (0, 0)),
        ],
        out_specs=pl.BlockSpec((TILE, H), lambda i: (i, 0)),
        out_shape=jax.ShapeDtypeStruct((B, H), jnp.float32),
        compiler_params=pltpu.CompilerParams(
            dimension_semantics=("parallel",),
        ),
    )(idx_col, idx_row, h_fast, h_slow, old, W_ih, W_hh, b_ih, b_hh)


def kernel(h_fast, h_slow, idx, A_states, W_ih, W_hh, b_ih, b_hh):
    flat_idx = idx.reshape(-1)
    old = _sc_gather(A_states, flat_idx)
    cp, new = _tc_fused(
        A_states, idx.reshape(B, 1), idx.reshape(1, B),
        h_fast.astype(jnp.bfloat16), h_slow.astype(jnp.bfloat16),
        old, W_ih, W_hh, b_ih.reshape(1, G), b_hh.reshape(1, G))
    out_ref = jax.new_ref(cp)
    _sc_scatter(out_ref, new, flat_idx)
    return out_ref[...]


# speedup vs baseline: 23.6979x; 1.1629x over previous
import jax
import jax.numpy as jnp
from jax import lax
from jax.experimental import pallas as pl
from jax.experimental.pallas import tpu as pltpu
from jax.experimental.pallas import tpu_sc as plsc

B = 4096
D = 64
K = 100000
H = 2 * D
G = 3 * H
EMA_M = 0.5

NC = 2
NS = 16
NW = NC * NS
BPW = B // NW

TILE = 512


def _sc_mesh():
    return plsc.VectorSubcoreMesh(
        core_axis_name="c", subcore_axis_name="s",
        num_cores=NC, num_subcores=NS)


def _gather_body(a_hbm, idx_hbm, old_hbm, idx_v, rows_v, sem):
    wid = lax.axis_index("s") * NC + lax.axis_index("c")
    base = wid * BPW
    pltpu.sync_copy(idx_hbm.at[pl.ds(base, BPW)], idx_v)
    pltpu.async_copy(a_hbm.at[idx_v], rows_v, sem).wait()
    pltpu.sync_copy(rows_v, old_hbm.at[pl.ds(base, BPW)])


def _sc_gather(a, flat_idx):
    return pl.kernel(
        _gather_body,
        out_type=jax.ShapeDtypeStruct((B, H), jnp.float32),
        mesh=_sc_mesh(),
        scratch_types=[
            pltpu.VMEM((BPW,), jnp.int32),
            pltpu.VMEM((BPW, H), jnp.float32),
            pltpu.SemaphoreType.DMA,
        ],
    )(a, flat_idx)


def _scatter_body(out_hbm, new_hbm, idx_hbm, idx_v, rows_v, sem):
    wid = lax.axis_index("s") * NC + lax.axis_index("c")
    base = wid * BPW
    pltpu.sync_copy(idx_hbm.at[pl.ds(base, BPW)], idx_v)
    pltpu.sync_copy(new_hbm.at[pl.ds(base, BPW)], rows_v)
    pltpu.async_copy(rows_v, out_hbm.at[idx_v], sem).wait()


def _sc_scatter(out_ref, new, flat_idx):
    return pl.kernel(
        _scatter_body,
        out_type=(),
        mesh=_sc_mesh(),
        scratch_types=[
            pltpu.VMEM((BPW,), jnp.int32),
            pltpu.VMEM((BPW, H), jnp.float32),
            pltpu.SemaphoreType.DMA,
        ],
    )(out_ref, new, flat_idx)


NSTEP = 10
CROWS = K // NSTEP
CTILE = 416


def _fused_body(a_ref, idxc_ref, idxr_ref, msg_ref, old_ref,
                wih_ref, whh_ref, bih_ref, bhh_ref, cp_ref, new_ref):
    cp_ref[...] = a_ref[...]

    eq = idxc_ref[...] == idxr_ref[...]
    e = eq.astype(jnp.bfloat16)
    cnt = jnp.sum(eq.astype(jnp.float32), axis=1, keepdims=True)
    dn = (((1,), (0,)), ((), ()))
    msum = lax.dot_general(e, msg_ref[...], dn,
                           preferred_element_type=jnp.float32)
    m = msum * (1.0 / (cnt + 1e-9))

    old = old_ref[...]
    dnt = (((1,), (1,)), ((), ()))
    gi = lax.dot_general(m, wih_ref[...], dnt) + bih_ref[...]
    gh = lax.dot_general(old, whh_ref[...], dnt) + bhh_ref[...]
    r = jax.nn.sigmoid(gi[:, :H] + gh[:, :H])
    z = jax.nn.sigmoid(gi[:, H:2 * H] + gh[:, H:2 * H])
    n = jnp.tanh(gi[:, 2 * H:] + r * gh[:, 2 * H:])
    hn = (1.0 - z) * n + z * old
    new_ref[...] = old * (1.0 - EMA_M) + hn * EMA_M


def _tc_fused(a, idx_col, idx_row, msg, old,
              W_ih, W_hh, b_ih, b_hh):
    return pl.pallas_call(
        _fused_body,
        grid=(NSTEP,),
        in_specs=[
            pl.BlockSpec((CROWS, H), lambda i: (i, 0)),
            pl.BlockSpec((CTILE, 1), lambda i: (i, 0)),
            pl.BlockSpec((1, B), lambda i: (0, 0)),
            pl.BlockSpec((B, H), lambda i: (0, 0)),
            pl.BlockSpec((CTILE, H), lambda i: (i, 0)),
            pl.BlockSpec((G, H), lambda i: (0, 0)),
            pl.BlockSpec((G, H), lambda i: (0, 0)),
            pl.BlockSpec((1, G), lambda i: (0, 0)),
            pl.BlockSpec((1, G), lambda i: (0, 0)),
        ],
        out_specs=[
            pl.BlockSpec((CROWS, H), lambda i: (i, 0)),
            pl.BlockSpec((CTILE, H), lambda i: (i, 0)),
        ],
        out_shape=[
            jax.ShapeDtypeStruct((K, H), jnp.float32),
            jax.ShapeDtypeStruct((B, H), jnp.float32),
        ],
        compiler_params=pltpu.CompilerParams(
            dimension_semantics=("parallel",),
        ),
    )(a, idx_col, idx_row, msg, old, W_ih, W_hh, b_ih, b_hh)


def _compute_body(idxc_ref, idxr_ref, hf_ref, hs_ref, old_ref,
                  wih_ref, whh_ref, bih_ref, bhh_ref, out_ref):
    eq = idxc_ref[...] == idxr_ref[...]
    e = eq.astype(jnp.bfloat16)
    cnt = jnp.sum(eq.astype(jnp.float32), axis=1, keepdims=True)
    dn = (((1,), (0,)), ((), ()))
    msf = lax.dot_general(e, hf_ref[...], dn,
                          preferred_element_type=jnp.float32)
    mss = lax.dot_general(e, hs_ref[...], dn,
                          preferred_element_type=jnp.float32)
    scale = 1.0 / (cnt + 1e-9)
    m = jnp.concatenate([msf * scale, mss * scale], axis=1)

    old = old_ref[...]
    dnt = (((1,), (1,)), ((), ()))
    gi = lax.dot_general(m, wih_ref[...], dnt) + bih_ref[...]
    gh = lax.dot_general(old, whh_ref[...], dnt) + bhh_ref[...]
    r = jax.nn.sigmoid(gi[:, :H] + gh[:, :H])
    z = jax.nn.sigmoid(gi[:, H:2 * H] + gh[:, H:2 * H])
    n = jnp.tanh(gi[:, 2 * H:] + r * gh[:, 2 * H:])
    hn = (1.0 - z) * n + z * old
    out_ref[...] = old * (1.0 - EMA_M) + hn * EMA_M


def _tc_compute(idx_col, idx_row, h_fast, h_slow, old, W_ih, W_hh, b_ih, b_hh):
    grid = (B // TILE,)
    return pl.pallas_call(
        _compute_body,
        grid=grid,
        in_specs=[
            pl.BlockSpec((TILE, 1), lambda i: (i, 0)),
            pl.BlockSpec((1, B), lambda i: (0, 0)),
            pl.BlockSpec((B, D), lambda i: (0, 0)),
            pl.BlockSpec((B, D), lambda i: (0, 0)),
            pl.BlockSpec((TILE, H), lambda i: (i, 0)),
            pl.BlockSpec((G, H), lambda i: (0, 0)),
            pl.BlockSpec((G, H), lambda i: (0, 0)),
            pl.BlockSpec((1, G), lambda i: (0, 0)),
            pl.BlockSpec((1, G), lambda i: (0, 0)),
        ],
        out_specs=pl.BlockSpec((TILE, H), lambda i: (i, 0)),
        out_shape=jax.ShapeDtypeStruct((B, H), jnp.float32),
        compiler_params=pltpu.CompilerParams(
            dimension_semantics=("parallel",),
        ),
    )(idx_col, idx_row, h_fast, h_slow, old, W_ih, W_hh, b_ih, b_hh)


def kernel(h_fast, h_slow, idx, A_states, W_ih, W_hh, b_ih, b_hh):
    flat_idx = idx.reshape(-1)
    old = _sc_gather(A_states, flat_idx)
    msg = jnp.concatenate([h_fast, h_slow], axis=1).astype(jnp.bfloat16)
    cp, new = _tc_fused(
        A_states, idx.reshape(B, 1), idx.reshape(1, B), msg,
        old, W_ih, W_hh, b_ih.reshape(1, G), b_hh.reshape(1, G))
    out_ref = jax.new_ref(cp)
    _sc_scatter(out_ref, new, flat_idx)
    return out_ref[...]
